# baseline (speedup 1.0000x reference)
import functools

import jax
import jax.numpy as jnp
from jax import lax
from jax.experimental import pallas as pl
from jax.experimental.pallas import tpu as pltpu
from jax.experimental.pallas import tpu_sc as plsc

VOCAB = 1000000
EMBED = 64
BATCH = 16384
NEG_K = 20

_info = plsc.get_sparse_core_info()
_NC, _NS = _info.num_cores, _info.num_subcores
_NW = _NC * _NS
_BPW = BATCH // _NW
_CHUNK = _BPW
_NEG_CHUNKS = NEG_K


def _body(center_hbm, pos_hbm, negf_hbm, in_hbm, out_hbm,
          emb_out, pos_out, neg_out,
          idx_v, rows_v, sem):
    wid = lax.axis_index("s") * _NC + lax.axis_index("c")
    base = wid * _BPW

    def one(idx_src, idx_off, table, dst, dst_off):
        pltpu.sync_copy(idx_src.at[pl.ds(idx_off, _CHUNK)], idx_v)
        pltpu.async_copy(table.at[idx_v], rows_v, sem).wait()
        pltpu.sync_copy(rows_v, dst.at[pl.ds(dst_off, _CHUNK)])

    one(center_hbm, base, in_hbm, emb_out, base)
    one(pos_hbm, base, out_hbm, pos_out, base)
    nbase = base * NEG_K
    for c in range(_NEG_CHUNKS):
        one(negf_hbm, nbase + c * _CHUNK, out_hbm, neg_out, nbase + c * _CHUNK)


@jax.jit
def _run(center_word, pos_word, neg_flat, in_embed, out_embed):
    mesh = plsc.VectorSubcoreMesh(core_axis_name="c", subcore_axis_name="s")
    f = pl.kernel(
        _body,
        mesh=mesh,
        out_type=(
            jax.ShapeDtypeStruct((BATCH, EMBED), jnp.float32),
            jax.ShapeDtypeStruct((BATCH, EMBED), jnp.float32),
            jax.ShapeDtypeStruct((BATCH * NEG_K, EMBED), jnp.float32),
        ),
        scratch_types=[
            pltpu.VMEM((_CHUNK,), jnp.int32),
            pltpu.VMEM((_CHUNK, EMBED), jnp.float32),
            pltpu.SemaphoreType.DMA,
        ],
        compiler_params=pltpu.CompilerParams(use_tc_tiling_on_sc=False),
    )
    return f(center_word, pos_word, neg_flat, in_embed, out_embed)


def kernel(center_word, pos_word, neg_word, in_embed, out_embed):
    neg_flat = neg_word.reshape(-1)
    emb, pos, negf = _run(center_word, pos_word, neg_flat, in_embed, out_embed)
    return (emb, pos, negf.reshape(BATCH, NEG_K, EMBED))
